# baseline (speedup 1.0000x reference)
import functools

import jax
import jax.numpy as jnp
from jax.experimental import pallas as pl
from jax.experimental.pallas import tpu as pltpu

_SCORE_THRESH = 0.05
_NMS_THRESH = 0.5
_DET = 300
_DETP = 304
_K = 1000
_KP = 1024
_N = 20000
_NP = 20480
_NB = 160
_TPAD = _KP + 128
_ONE_BITS = 0x3F800000


def _gauss_params(cx, cy, w, h, ang):
    c, s = jnp.cos(ang), jnp.sin(ang)
    w2, h2 = (w * w) / 12.0, (h * h) / 12.0
    a = w2 * c * c + h2 * s * s
    b = w2 * s * s + h2 * c * c
    cc = (w2 - h2) * c * s
    return cx, cy, a, b, cc


def _body(feat_ref, scores_ref,
          ob_ref, os_ref, ol_ref,
          s_ref, p_ref, tab_ref, sel_ref, offs_ref, keep_ref):
    f32 = jnp.float32
    sc = scores_ref[0]
    key = jnp.where(sc > _SCORE_THRESH,
                    jax.lax.bitcast_convert_type(sc, jnp.int32),
                    jnp.int32(-1))
    gidx = (jax.lax.broadcasted_iota(jnp.int32, (_NB, 128), 0) * 128
            + jax.lax.broadcasted_iota(jnp.int32, (_NB, 128), 1))

    def _val_step(_, lohi):
        lo, hi = lohi
        mid = lo + (hi - lo) // 2
        cnt = jnp.sum((key >= mid).astype(f32))
        big = cnt >= _K
        return jnp.where(big, mid, lo), jnp.where(big, hi, mid)

    tau, _ = jax.lax.fori_loop(
        0, 31, _val_step, (jnp.int32(-1), jnp.int32(_ONE_BITS)))
    eq = key == tau
    cnt_gt = jnp.sum((key > tau).astype(f32))
    m_need = f32(_K) - cnt_gt

    def _idx_step(_, lohi):
        lo, hi = lohi
        mid = lo + (hi - lo) // 2
        cnt = jnp.sum((eq & (gidx < mid)).astype(f32))
        ok = cnt <= m_need
        return jnp.where(ok, mid, lo), jnp.where(ok, hi, mid)

    tcut, _ = jax.lax.fori_loop(
        0, 16, _idx_step, (jnp.int32(0), jnp.int32(32768)))
    sel = ((key > tau) | (eq & (gidx < tcut))).astype(f32)
    sel_ref[...] = sel

    cnts = jnp.sum(sel, axis=1, keepdims=True)
    lstrict = (jax.lax.broadcasted_iota(jnp.int32, (_NB, _NB), 1)
               < jax.lax.broadcasted_iota(jnp.int32, (_NB, _NB), 0)).astype(f32)
    offs_ref[...] = jnp.minimum(jax.lax.dot_general(
        lstrict, cnts, (((1,), (0,)), ((), ())),
        preferred_element_type=f32), f32(_K))
    tri = (jax.lax.broadcasted_iota(jnp.int32, (128, 128), 0)
           <= jax.lax.broadcasted_iota(jnp.int32, (128, 128), 1)).astype(f32)
    riota = jax.lax.broadcasted_iota(jnp.int32, (128, 128), 0).astype(f32)
    dn = (((1,), (1,)), ((), ()))

    def _gather_step(t, carry):
        m = sel_ref[pl.ds(t, 1), :]
        csum = jax.lax.dot_general(m, tri, (((1,), (0,)), ((), ())),
                                   preferred_element_type=f32)
        g = jnp.where(m > 0.5, csum - 1.0, -1.0)
        ct = (riota == g).astype(f32)
        dt = feat_ref[0, :, pl.ds(t, 1), :].reshape(9, 128)
        compact = jax.lax.dot_general(
            ct, dt, dn, preferred_element_type=f32,
            precision=jax.lax.Precision.HIGHEST)
        off = offs_ref[t, 0].astype(jnp.int32)
        tab_ref[pl.ds(off, 128), :] = compact
        return carry

    jax.lax.fori_loop(0, _NB, _gather_step, 0)
    tab_ref[_K:, :] = jnp.zeros((_TPAD - _K, 9), f32)

    tab = tab_ref[0:_KP, :]
    tabt = jnp.transpose(tab)
    cxc, cyc, wc, hc, ac = (tab[:, i:i + 1] for i in range(5))
    scc, labc, idxc = tab[:, 5:6], tab[:, 6:7], tab[:, 7:8]
    cxr, cyr, wr, hr, ar = (tabt[i:i + 1, :] for i in range(5))
    scr, labr, idxr = tabt[5:6, :], tabt[6:7, :], tabt[7:8, :]

    x1, y1, a1, b1, c1 = _gauss_params(cxc, cyc, wc, hc, ac)
    x2, y2, a2, b2, c2 = _gauss_params(cxr, cyr, wr, hr, ar)
    eps = 1e-7
    dx = x1 - x2
    dy = y1 - y2
    ab = (a1 + a2) * (b1 + b2) - (c1 + c2) ** 2
    t1 = 0.25 * ((a1 + a2) * dy * dy + (b1 + b2) * dx * dx) / (ab + eps)
    t2 = 0.5 * ((c1 + c2) * (-dx) * dy) / (ab + eps)
    d1 = jnp.clip(a1 * b1 - c1 * c1, eps, None)
    d2 = jnp.clip(a2 * b2 - c2 * c2, eps, None)
    t3 = 0.5 * jnp.log(ab / (4.0 * jnp.sqrt(d1 * d2) + eps) + eps)
    bd = jnp.clip(t1 + t2 + t3, eps, 100.0)
    hd = jnp.sqrt(1.0 - jnp.exp(-bd) + eps)
    iou = 1.0 - hd

    prio = (scc > scr) | ((scc == scr) & (idxc < idxr))
    p_ref[...] = prio.astype(f32)
    s_ref[...] = ((iou > _NMS_THRESH) & (labc == labr) & prio).astype(f32)

    valid = (scr > _SCORE_THRESH).astype(f32)
    keep_ref[...] = valid
    dn_mv = (((1,), (0,)), ((), ()))

    def _nms_cond(changed):
        return changed

    def _nms_step(_):
        old = keep_ref[...]
        hit = jax.lax.dot_general(old, s_ref[...], dn_mv,
                                  preferred_element_type=f32)
        new = valid * (hit < 0.5).astype(f32)
        keep_ref[...] = new
        return jnp.any(new != old)

    jax.lax.while_loop(_nms_cond, _nms_step, jnp.bool_(True))

    kept = keep_ref[...]
    rank = jax.lax.dot_general(kept, p_ref[...], dn_mv,
                               preferred_element_type=f32)
    piota = jax.lax.broadcasted_iota(jnp.int32, (_DETP, _KP), 0).astype(f32)
    oh = ((piota == rank) & (kept > 0.5)).astype(f32)
    out = jax.lax.dot_general(oh, tab, dn_mv,
                              preferred_element_type=f32, precision=jax.lax.Precision.HIGHEST)
    outt = jax.lax.dot_general(tabt, oh, dn,
                               preferred_element_type=f32, precision=jax.lax.Precision.HIGHEST)
    ob_ref[...] = out[:_DET, 0:5].reshape(1, _DET, 5)
    os_ref[...] = outt[5:6, :_DET].reshape(1, 1, _DET)
    ind = outt[8:9, :_DET]
    ol_ref[...] = jnp.where(ind > 0.5, outt[6:7, :_DET],
                            -1.0).astype(jnp.int32).reshape(1, 1, _DET)


@jax.jit
def kernel(boxes, scores, labels):
    b = boxes.shape[0]
    f32 = jnp.float32
    pad = _NP - _N
    sc = jnp.pad(scores, ((0, 0), (0, pad)),
                 constant_values=-1.0).reshape(b, _NB, 128)
    bx = jnp.pad(boxes, ((0, 0), (0, pad), (0, 0)))
    bx = jnp.transpose(bx, (0, 2, 1))
    scp = jnp.pad(scores, ((0, 0), (0, pad)))[:, None, :]
    lbp = jnp.pad(labels, ((0, 0), (0, pad))).astype(f32)[:, None, :]
    idx = jnp.broadcast_to(jnp.arange(_NP, dtype=f32), (b, 1, _NP))
    one = jnp.ones((b, 1, _NP), f32)
    feat = jnp.concatenate([bx, scp, lbp, idx, one],
                           axis=1).reshape(b, 9, _NB, 128)

    grid = (b,)
    out = pl.pallas_call(
        _body,
        grid=grid,
        in_specs=[
            pl.BlockSpec((1, 9, _NB, 128), lambda i: (i, 0, 0, 0)),
            pl.BlockSpec((1, _NB, 128), lambda i: (i, 0, 0)),
        ],
        out_specs=[
            pl.BlockSpec((1, _DET, 5), lambda i: (i, 0, 0)),
            pl.BlockSpec((1, 1, _DET), lambda i: (i, 0, 0)),
            pl.BlockSpec((1, 1, _DET), lambda i: (i, 0, 0)),
        ],
        out_shape=[
            jax.ShapeDtypeStruct((b, _DET, 5), f32),
            jax.ShapeDtypeStruct((b, 1, _DET), f32),
            jax.ShapeDtypeStruct((b, 1, _DET), jnp.int32),
        ],
        scratch_shapes=[
            pltpu.VMEM((_KP, _KP), f32),
            pltpu.VMEM((_KP, _KP), f32),
            pltpu.VMEM((_TPAD, 9), f32),
            pltpu.VMEM((_NB, 128), f32),
            pltpu.VMEM((_NB, 1), f32),
            pltpu.VMEM((1, _KP), f32),
        ],
    )(feat, sc)
    return out[0], out[1].reshape(b, _DET), out[2].reshape(b, _DET)


# speedup vs baseline: 1.4980x; 1.4980x over previous
import functools

import jax
import jax.numpy as jnp
from jax.experimental import pallas as pl
from jax.experimental.pallas import tpu as pltpu

_SCORE_THRESH = 0.05
_NMS_THRESH = 0.5
_DET = 300
_DETP = 304
_K = 1000
_KP = 1024
_N = 20000
_NP = 20480
_NB = 160
_TPAD = _KP + 128
_ONE_BITS = 0x3F800000


def _gauss_params(cx, cy, w, h, ang):
    c, s = jnp.cos(ang), jnp.sin(ang)
    w2, h2 = (w * w) / 12.0, (h * h) / 12.0
    a = w2 * c * c + h2 * s * s
    b = w2 * s * s + h2 * c * c
    cc = (w2 - h2) * c * s
    return cx, cy, a, b, cc


def _body(feat_ref, scores_ref,
          ob_ref, os_ref, ol_ref,
          s_ref, p_ref, tab_ref, sel_ref, offs_ref, keep_ref):
    f32 = jnp.float32
    sc = scores_ref[0]
    key = jnp.where(sc > _SCORE_THRESH,
                    jax.lax.bitcast_convert_type(sc, jnp.int32),
                    jnp.int32(-1))
    gidx = (jax.lax.broadcasted_iota(jnp.int32, (_NB, 128), 0) * 128
            + jax.lax.broadcasted_iota(jnp.int32, (_NB, 128), 1))

    def _val_step(_, lohi):
        lo, hi = lohi
        mid = lo + (hi - lo) // 2
        cnt = jnp.sum((key >= mid).astype(f32))
        big = cnt >= _K
        return jnp.where(big, mid, lo), jnp.where(big, hi, mid)

    tau, _ = jax.lax.fori_loop(
        0, 31, _val_step, (jnp.int32(-1), jnp.int32(_ONE_BITS)))
    eq = key == tau
    cnt_gt = jnp.sum((key > tau).astype(f32))
    m_need = f32(_K) - cnt_gt

    def _idx_step(_, lohi):
        lo, hi = lohi
        mid = lo + (hi - lo) // 2
        cnt = jnp.sum((eq & (gidx < mid)).astype(f32))
        ok = cnt <= m_need
        return jnp.where(ok, mid, lo), jnp.where(ok, hi, mid)

    tcut, _ = jax.lax.fori_loop(
        0, 16, _idx_step, (jnp.int32(0), jnp.int32(32768)))
    sel = ((key > tau) | (eq & (gidx < tcut))).astype(f32)
    sel_ref[...] = sel

    cnts = jnp.sum(sel, axis=1, keepdims=True)
    lstrict = (jax.lax.broadcasted_iota(jnp.int32, (_NB, _NB), 1)
               < jax.lax.broadcasted_iota(jnp.int32, (_NB, _NB), 0)).astype(f32)
    offs_ref[...] = jnp.minimum(jax.lax.dot_general(
        lstrict, cnts, (((1,), (0,)), ((), ())),
        preferred_element_type=f32), f32(_K))
    tri = (jax.lax.broadcasted_iota(jnp.int32, (128, 128), 0)
           <= jax.lax.broadcasted_iota(jnp.int32, (128, 128), 1)).astype(f32)
    riota = jax.lax.broadcasted_iota(jnp.int32, (128, 128), 0).astype(f32)
    dn = (((1,), (1,)), ((), ()))

    def _gather_step(t, carry):
        m = sel_ref[pl.ds(t, 1), :]
        csum = jax.lax.dot_general(m, tri, (((1,), (0,)), ((), ())),
                                   preferred_element_type=f32)
        g = jnp.where(m > 0.5, csum - 1.0, -1.0)
        ct = (riota == g).astype(f32)
        dt = feat_ref[0, :, pl.ds(t, 1), :].reshape(9, 128)
        compact = jax.lax.dot_general(
            ct, dt, dn, preferred_element_type=f32,
            precision=jax.lax.Precision.HIGHEST)
        off = offs_ref[t, 0].astype(jnp.int32)
        tab_ref[pl.ds(off, 128), :] = compact
        return carry

    jax.lax.fori_loop(0, _NB, _gather_step, 0, unroll=4)
    tab_ref[_K:, :] = jnp.zeros((_TPAD - _K, 9), f32)

    tab = tab_ref[0:_KP, :]
    tabt = jnp.transpose(tab)
    cxc, cyc, wc, hc, ac = (tab[:, i:i + 1] for i in range(5))
    scc, labc, idxc = tab[:, 5:6], tab[:, 6:7], tab[:, 7:8]
    cxr, cyr, wr, hr, ar = (tabt[i:i + 1, :] for i in range(5))
    scr, labr, idxr = tabt[5:6, :], tabt[6:7, :], tabt[7:8, :]

    x1, y1, a1, b1, c1 = _gauss_params(cxc, cyc, wc, hc, ac)
    x2, y2, a2, b2, c2 = _gauss_params(cxr, cyr, wr, hr, ar)
    eps = 1e-7
    dx = x1 - x2
    dy = y1 - y2
    ab = (a1 + a2) * (b1 + b2) - (c1 + c2) ** 2
    t1 = 0.25 * ((a1 + a2) * dy * dy + (b1 + b2) * dx * dx) / (ab + eps)
    t2 = 0.5 * ((c1 + c2) * (-dx) * dy) / (ab + eps)
    d1 = jnp.clip(a1 * b1 - c1 * c1, eps, None)
    d2 = jnp.clip(a2 * b2 - c2 * c2, eps, None)
    t3 = 0.5 * jnp.log(ab / (4.0 * jnp.sqrt(d1 * d2) + eps) + eps)
    bd = jnp.clip(t1 + t2 + t3, eps, 100.0)
    hd = jnp.sqrt(1.0 - jnp.exp(-bd) + eps)
    iou = 1.0 - hd

    prio = (scc > scr) | ((scc == scr) & (idxc < idxr))
    p_ref[...] = prio.astype(f32)
    s_ref[...] = ((iou > _NMS_THRESH) & (labc == labr) & prio).astype(f32)

    valid = (scr > _SCORE_THRESH).astype(f32)
    keep_ref[...] = valid
    dn_mv = (((1,), (0,)), ((), ()))

    def _nms_cond(changed):
        return changed

    def _nms_step(_):
        old = keep_ref[...]
        hit = jax.lax.dot_general(old, s_ref[...], dn_mv,
                                  preferred_element_type=f32)
        new = valid * (hit < 0.5).astype(f32)
        keep_ref[...] = new
        return jnp.any(new != old)

    jax.lax.while_loop(_nms_cond, _nms_step, jnp.bool_(True))

    kept = keep_ref[...]
    rank = jax.lax.dot_general(kept, p_ref[...], dn_mv,
                               preferred_element_type=f32)
    piota = jax.lax.broadcasted_iota(jnp.int32, (_DETP, _KP), 0).astype(f32)
    oh = ((piota == rank) & (kept > 0.5)).astype(f32)
    out = jax.lax.dot_general(oh, tab, dn_mv,
                              preferred_element_type=f32, precision=jax.lax.Precision.HIGHEST)
    outt = jax.lax.dot_general(tabt, oh, dn,
                               preferred_element_type=f32, precision=jax.lax.Precision.HIGHEST)
    ob_ref[...] = out[:_DET, 0:5].reshape(1, _DET, 5)
    os_ref[...] = outt[5:6, :_DET].reshape(1, 1, _DET)
    ind = outt[8:9, :_DET]
    ol_ref[...] = jnp.where(ind > 0.5, outt[6:7, :_DET],
                            -1.0).astype(jnp.int32).reshape(1, 1, _DET)


@jax.jit
def kernel(boxes, scores, labels):
    b = boxes.shape[0]
    f32 = jnp.float32
    pad = _NP - _N
    sc = jnp.pad(scores, ((0, 0), (0, pad)),
                 constant_values=-1.0).reshape(b, _NB, 128)
    bx = jnp.pad(boxes, ((0, 0), (0, pad), (0, 0)))
    bx = jnp.transpose(bx, (0, 2, 1))
    scp = jnp.pad(scores, ((0, 0), (0, pad)))[:, None, :]
    lbp = jnp.pad(labels, ((0, 0), (0, pad))).astype(f32)[:, None, :]
    idx = jnp.broadcast_to(jnp.arange(_NP, dtype=f32), (b, 1, _NP))
    one = jnp.ones((b, 1, _NP), f32)
    feat = jnp.concatenate([bx, scp, lbp, idx, one],
                           axis=1).reshape(b, 9, _NB, 128)

    grid = (b,)
    out = pl.pallas_call(
        _body,
        grid=grid,
        in_specs=[
            pl.BlockSpec((1, 9, _NB, 128), lambda i: (i, 0, 0, 0)),
            pl.BlockSpec((1, _NB, 128), lambda i: (i, 0, 0)),
        ],
        out_specs=[
            pl.BlockSpec((1, _DET, 5), lambda i: (i, 0, 0)),
            pl.BlockSpec((1, 1, _DET), lambda i: (i, 0, 0)),
            pl.BlockSpec((1, 1, _DET), lambda i: (i, 0, 0)),
        ],
        out_shape=[
            jax.ShapeDtypeStruct((b, _DET, 5), f32),
            jax.ShapeDtypeStruct((b, 1, _DET), f32),
            jax.ShapeDtypeStruct((b, 1, _DET), jnp.int32),
        ],
        scratch_shapes=[
            pltpu.VMEM((_KP, _KP), f32),
            pltpu.VMEM((_KP, _KP), f32),
            pltpu.VMEM((_TPAD, 9), f32),
            pltpu.VMEM((_NB, 128), f32),
            pltpu.VMEM((_NB, 1), f32),
            pltpu.VMEM((1, _KP), f32),
        ],
    )(feat, sc)
    return out[0], out[1].reshape(b, _DET), out[2].reshape(b, _DET)


# speedup vs baseline: 1.5773x; 1.0530x over previous
import functools

import jax
import jax.numpy as jnp
from jax.experimental import pallas as pl
from jax.experimental.pallas import tpu as pltpu

_SCORE_THRESH = 0.05
_NMS_THRESH = 0.5
_DET = 300
_DETP = 304
_K = 1000
_KP = 1024
_N = 20000
_NP = 20480
_NB = 160
_TPAD = _KP + 128
_ONE_BITS = 0x3F800000


def _gauss_params(cx, cy, w, h, ang):
    c, s = jnp.cos(ang), jnp.sin(ang)
    w2, h2 = (w * w) / 12.0, (h * h) / 12.0
    a = w2 * c * c + h2 * s * s
    b = w2 * s * s + h2 * c * c
    cc = (w2 - h2) * c * s
    return cx, cy, a, b, cc


def _body(feat_ref, scores_ref,
          ob_ref, os_ref, ol_ref,
          s_ref, p_ref, tab_ref, sel_ref, offs_ref, keep_ref):
    f32 = jnp.float32
    sc = scores_ref[0]
    key = jnp.where(sc > _SCORE_THRESH,
                    jax.lax.bitcast_convert_type(sc, jnp.int32),
                    jnp.int32(-1))
    gidx = (jax.lax.broadcasted_iota(jnp.int32, (_NB, 128), 0) * 128
            + jax.lax.broadcasted_iota(jnp.int32, (_NB, 128), 1))

    def _val_step(_, lohi):
        lo, hi = lohi
        mid = lo + (hi - lo) // 2
        cnt = jnp.sum((key >= mid).astype(f32))
        big = cnt >= _K
        return jnp.where(big, mid, lo), jnp.where(big, hi, mid)

    tau, _ = jax.lax.fori_loop(
        0, 31, _val_step, (jnp.int32(-1), jnp.int32(_ONE_BITS)))
    eq = key == tau
    cnt_gt = jnp.sum((key > tau).astype(f32))
    m_need = f32(_K) - cnt_gt

    def _idx_step(_, lohi):
        lo, hi = lohi
        mid = lo + (hi - lo) // 2
        cnt = jnp.sum((eq & (gidx < mid)).astype(f32))
        ok = cnt <= m_need
        return jnp.where(ok, mid, lo), jnp.where(ok, hi, mid)

    tcut, _ = jax.lax.fori_loop(
        0, 16, _idx_step, (jnp.int32(0), jnp.int32(32768)))
    sel = ((key > tau) | (eq & (gidx < tcut))).astype(f32)
    sel_ref[...] = sel

    cnts = jnp.sum(sel, axis=1, keepdims=True)
    lstrict = (jax.lax.broadcasted_iota(jnp.int32, (_NB, _NB), 1)
               < jax.lax.broadcasted_iota(jnp.int32, (_NB, _NB), 0)).astype(f32)
    offs_ref[...] = jnp.minimum(jax.lax.dot_general(
        lstrict, cnts, (((1,), (0,)), ((), ())),
        preferred_element_type=f32), f32(_K))
    tri = (jax.lax.broadcasted_iota(jnp.int32, (128, 128), 0)
           <= jax.lax.broadcasted_iota(jnp.int32, (128, 128), 1)).astype(f32)
    riota = jax.lax.broadcasted_iota(jnp.int32, (128, 128), 0).astype(f32)
    dn = (((1,), (1,)), ((), ()))

    def _gather_step(t, carry):
        m = sel_ref[pl.ds(t, 1), :]
        csum = jax.lax.dot_general(m, tri, (((1,), (0,)), ((), ())),
                                   preferred_element_type=f32)
        g = jnp.where(m > 0.5, csum - 1.0, -1.0)
        ct = (riota == g).astype(f32)
        dt = feat_ref[0, :, pl.ds(t, 1), :].reshape(9, 128)
        compact = jax.lax.dot_general(
            ct, dt, dn, preferred_element_type=f32,
            precision=jax.lax.Precision.HIGHEST)
        off = offs_ref[t, 0].astype(jnp.int32)
        tab_ref[pl.ds(off, 128), :] = compact
        return carry

    jax.lax.fori_loop(0, _NB, _gather_step, 0, unroll=8)
    tab_ref[_K:, :] = jnp.zeros((_TPAD - _K, 9), f32)

    tab = tab_ref[0:_KP, :]
    tabt = jnp.transpose(tab)
    cxc, cyc, wc, hc, ac = (tab[:, i:i + 1] for i in range(5))
    scc, labc, idxc = tab[:, 5:6], tab[:, 6:7], tab[:, 7:8]
    cxr, cyr, wr, hr, ar = (tabt[i:i + 1, :] for i in range(5))
    scr, labr, idxr = tabt[5:6, :], tabt[6:7, :], tabt[7:8, :]

    x1, y1, a1, b1, c1 = _gauss_params(cxc, cyc, wc, hc, ac)
    x2, y2, a2, b2, c2 = _gauss_params(cxr, cyr, wr, hr, ar)
    eps = 1e-7
    dx = x1 - x2
    dy = y1 - y2
    ab = (a1 + a2) * (b1 + b2) - (c1 + c2) ** 2
    t1 = 0.25 * ((a1 + a2) * dy * dy + (b1 + b2) * dx * dx) / (ab + eps)
    t2 = 0.5 * ((c1 + c2) * (-dx) * dy) / (ab + eps)
    d1 = jnp.clip(a1 * b1 - c1 * c1, eps, None)
    d2 = jnp.clip(a2 * b2 - c2 * c2, eps, None)
    t3 = 0.5 * jnp.log(ab / (4.0 * jnp.sqrt(d1 * d2) + eps) + eps)
    bd = jnp.clip(t1 + t2 + t3, eps, 100.0)
    hd = jnp.sqrt(1.0 - jnp.exp(-bd) + eps)
    iou = 1.0 - hd

    prio = (scc > scr) | ((scc == scr) & (idxc < idxr))
    p_ref[...] = prio.astype(f32)
    s_ref[...] = ((iou > _NMS_THRESH) & (labc == labr) & prio).astype(f32)

    valid = (scr > _SCORE_THRESH).astype(f32)
    keep_ref[...] = valid
    dn_mv = (((1,), (0,)), ((), ()))

    def _nms_cond(changed):
        return changed

    def _nms_step(_):
        old = keep_ref[...]
        hit = jax.lax.dot_general(old, s_ref[...], dn_mv,
                                  preferred_element_type=f32)
        new = valid * (hit < 0.5).astype(f32)
        keep_ref[...] = new
        return jnp.any(new != old)

    jax.lax.while_loop(_nms_cond, _nms_step, jnp.bool_(True))

    kept = keep_ref[...]
    rank = jax.lax.dot_general(kept, p_ref[...], dn_mv,
                               preferred_element_type=f32)
    piota = jax.lax.broadcasted_iota(jnp.int32, (_DETP, _KP), 0).astype(f32)
    oh = ((piota == rank) & (kept > 0.5)).astype(f32)
    out = jax.lax.dot_general(oh, tab, dn_mv,
                              preferred_element_type=f32, precision=jax.lax.Precision.HIGHEST)
    outt = jax.lax.dot_general(tabt, oh, dn,
                               preferred_element_type=f32, precision=jax.lax.Precision.HIGHEST)
    ob_ref[...] = out[:_DET, 0:5].reshape(1, _DET, 5)
    os_ref[...] = outt[5:6, :_DET].reshape(1, 1, _DET)
    ind = outt[8:9, :_DET]
    ol_ref[...] = jnp.where(ind > 0.5, outt[6:7, :_DET],
                            -1.0).astype(jnp.int32).reshape(1, 1, _DET)


@jax.jit
def kernel(boxes, scores, labels):
    b = boxes.shape[0]
    f32 = jnp.float32
    pad = _NP - _N
    sc = jnp.pad(scores, ((0, 0), (0, pad)),
                 constant_values=-1.0).reshape(b, _NB, 128)
    bx = jnp.pad(boxes, ((0, 0), (0, pad), (0, 0)))
    bx = jnp.transpose(bx, (0, 2, 1))
    scp = jnp.pad(scores, ((0, 0), (0, pad)))[:, None, :]
    lbp = jnp.pad(labels, ((0, 0), (0, pad))).astype(f32)[:, None, :]
    idx = jnp.broadcast_to(jnp.arange(_NP, dtype=f32), (b, 1, _NP))
    one = jnp.ones((b, 1, _NP), f32)
    feat = jnp.concatenate([bx, scp, lbp, idx, one],
                           axis=1).reshape(b, 9, _NB, 128)

    grid = (b,)
    out = pl.pallas_call(
        _body,
        grid=grid,
        in_specs=[
            pl.BlockSpec((1, 9, _NB, 128), lambda i: (i, 0, 0, 0)),
            pl.BlockSpec((1, _NB, 128), lambda i: (i, 0, 0)),
        ],
        out_specs=[
            pl.BlockSpec((1, _DET, 5), lambda i: (i, 0, 0)),
            pl.BlockSpec((1, 1, _DET), lambda i: (i, 0, 0)),
            pl.BlockSpec((1, 1, _DET), lambda i: (i, 0, 0)),
        ],
        out_shape=[
            jax.ShapeDtypeStruct((b, _DET, 5), f32),
            jax.ShapeDtypeStruct((b, 1, _DET), f32),
            jax.ShapeDtypeStruct((b, 1, _DET), jnp.int32),
        ],
        scratch_shapes=[
            pltpu.VMEM((_KP, _KP), f32),
            pltpu.VMEM((_KP, _KP), f32),
            pltpu.VMEM((_TPAD, 9), f32),
            pltpu.VMEM((_NB, 128), f32),
            pltpu.VMEM((_NB, 1), f32),
            pltpu.VMEM((1, _KP), f32),
        ],
    )(feat, sc)
    return out[0], out[1].reshape(b, _DET), out[2].reshape(b, _DET)


# speedup vs baseline: 1.6205x; 1.0274x over previous
import functools

import jax
import jax.numpy as jnp
from jax.experimental import pallas as pl
from jax.experimental.pallas import tpu as pltpu

_SCORE_THRESH = 0.05
_NMS_THRESH = 0.5
_DET = 300
_DETP = 304
_K = 1000
_KP = 1024
_N = 20000
_NP = 20480
_NB = 160
_TPAD = _KP + 128
_ONE_BITS = 0x3F800000


def _gauss_params(cx, cy, w, h, ang):
    c, s = jnp.cos(ang), jnp.sin(ang)
    w2, h2 = (w * w) / 12.0, (h * h) / 12.0
    a = w2 * c * c + h2 * s * s
    b = w2 * s * s + h2 * c * c
    cc = (w2 - h2) * c * s
    return cx, cy, a, b, cc


def _body(feat_ref, scores_ref,
          ob_ref, os_ref, ol_ref,
          s_ref, p_ref, tab_ref, sel_ref, offs_ref, keep_ref):
    f32 = jnp.float32
    sc = scores_ref[0]
    key = jnp.where(sc > _SCORE_THRESH,
                    jax.lax.bitcast_convert_type(sc, jnp.int32),
                    jnp.int32(-1))
    gidx = (jax.lax.broadcasted_iota(jnp.int32, (_NB, 128), 0) * 128
            + jax.lax.broadcasted_iota(jnp.int32, (_NB, 128), 1))

    def _val_step(_, lohi):
        lo, hi = lohi
        mid = lo + (hi - lo) // 2
        cnt = jnp.sum((key >= mid).astype(f32))
        big = cnt >= _K
        return jnp.where(big, mid, lo), jnp.where(big, hi, mid)

    tau, _ = jax.lax.fori_loop(
        0, 31, _val_step, (jnp.int32(-1), jnp.int32(_ONE_BITS)))
    eq = key == tau
    cnt_gt = jnp.sum((key > tau).astype(f32))
    m_need = f32(_K) - cnt_gt

    def _idx_step(_, lohi):
        lo, hi = lohi
        mid = lo + (hi - lo) // 2
        cnt = jnp.sum((eq & (gidx < mid)).astype(f32))
        ok = cnt <= m_need
        return jnp.where(ok, mid, lo), jnp.where(ok, hi, mid)

    tcut, _ = jax.lax.fori_loop(
        0, 16, _idx_step, (jnp.int32(0), jnp.int32(32768)))
    sel = ((key > tau) | (eq & (gidx < tcut))).astype(f32)
    sel_ref[...] = sel

    cnts = jnp.sum(sel, axis=1, keepdims=True)
    lstrict = (jax.lax.broadcasted_iota(jnp.int32, (_NB, _NB), 1)
               < jax.lax.broadcasted_iota(jnp.int32, (_NB, _NB), 0)).astype(f32)
    offs_ref[...] = jnp.minimum(jax.lax.dot_general(
        lstrict, cnts, (((1,), (0,)), ((), ())),
        preferred_element_type=f32), f32(_K))
    tri = (jax.lax.broadcasted_iota(jnp.int32, (128, 128), 0)
           <= jax.lax.broadcasted_iota(jnp.int32, (128, 128), 1)).astype(f32)
    riota = jax.lax.broadcasted_iota(jnp.int32, (128, 128), 0).astype(f32)
    dn = (((1,), (1,)), ((), ()))

    def _gather_step(t, carry):
        m = sel_ref[pl.ds(t, 1), :]
        csum = jax.lax.dot_general(m, tri, (((1,), (0,)), ((), ())),
                                   preferred_element_type=f32)
        g = jnp.where(m > 0.5, csum - 1.0, -1.0)
        ct = (riota == g).astype(f32)
        dt = feat_ref[0, :, pl.ds(t, 1), :].reshape(9, 128)
        compact = jax.lax.dot_general(
            ct, dt, dn, preferred_element_type=f32,
            precision=jax.lax.Precision.HIGHEST)
        off = offs_ref[t, 0].astype(jnp.int32)
        tab_ref[pl.ds(off, 128), :] = compact
        return carry

    jax.lax.fori_loop(0, _NB, _gather_step, 0, unroll=16)
    tab_ref[_K:, :] = jnp.zeros((_TPAD - _K, 9), f32)

    tab = tab_ref[0:_KP, :]
    tabt = jnp.transpose(tab)
    cxc, cyc, wc, hc, ac = (tab[:, i:i + 1] for i in range(5))
    scc, labc, idxc = tab[:, 5:6], tab[:, 6:7], tab[:, 7:8]
    cxr, cyr, wr, hr, ar = (tabt[i:i + 1, :] for i in range(5))
    scr, labr, idxr = tabt[5:6, :], tabt[6:7, :], tabt[7:8, :]

    x1, y1, a1, b1, c1 = _gauss_params(cxc, cyc, wc, hc, ac)
    x2, y2, a2, b2, c2 = _gauss_params(cxr, cyr, wr, hr, ar)
    eps = 1e-7
    dx = x1 - x2
    dy = y1 - y2
    ab = (a1 + a2) * (b1 + b2) - (c1 + c2) ** 2
    t1 = 0.25 * ((a1 + a2) * dy * dy + (b1 + b2) * dx * dx) / (ab + eps)
    t2 = 0.5 * ((c1 + c2) * (-dx) * dy) / (ab + eps)
    d1 = jnp.clip(a1 * b1 - c1 * c1, eps, None)
    d2 = jnp.clip(a2 * b2 - c2 * c2, eps, None)
    t3 = 0.5 * jnp.log(ab / (4.0 * jnp.sqrt(d1 * d2) + eps) + eps)
    bd = jnp.clip(t1 + t2 + t3, eps, 100.0)
    hd = jnp.sqrt(1.0 - jnp.exp(-bd) + eps)
    iou = 1.0 - hd

    prio = (scc > scr) | ((scc == scr) & (idxc < idxr))
    p_ref[...] = prio.astype(f32)
    s_ref[...] = ((iou > _NMS_THRESH) & (labc == labr) & prio).astype(f32)

    valid = (scr > _SCORE_THRESH).astype(f32)
    keep_ref[...] = valid
    dn_mv = (((1,), (0,)), ((), ()))

    def _nms_cond(changed):
        return changed

    def _nms_step(_):
        old = keep_ref[...]
        hit = jax.lax.dot_general(old, s_ref[...], dn_mv,
                                  preferred_element_type=f32)
        new = valid * (hit < 0.5).astype(f32)
        keep_ref[...] = new
        return jnp.any(new != old)

    jax.lax.while_loop(_nms_cond, _nms_step, jnp.bool_(True))

    kept = keep_ref[...]
    rank = jax.lax.dot_general(kept, p_ref[...], dn_mv,
                               preferred_element_type=f32)
    piota = jax.lax.broadcasted_iota(jnp.int32, (_DETP, _KP), 0).astype(f32)
    oh = ((piota == rank) & (kept > 0.5)).astype(f32)
    out = jax.lax.dot_general(oh, tab, dn_mv,
                              preferred_element_type=f32, precision=jax.lax.Precision.HIGHEST)
    outt = jax.lax.dot_general(tabt, oh, dn,
                               preferred_element_type=f32, precision=jax.lax.Precision.HIGHEST)
    ob_ref[...] = out[:_DET, 0:5].reshape(1, _DET, 5)
    os_ref[...] = outt[5:6, :_DET].reshape(1, 1, _DET)
    ind = outt[8:9, :_DET]
    ol_ref[...] = jnp.where(ind > 0.5, outt[6:7, :_DET],
                            -1.0).astype(jnp.int32).reshape(1, 1, _DET)


@jax.jit
def kernel(boxes, scores, labels):
    b = boxes.shape[0]
    f32 = jnp.float32
    pad = _NP - _N
    sc = jnp.pad(scores, ((0, 0), (0, pad)),
                 constant_values=-1.0).reshape(b, _NB, 128)
    bx = jnp.pad(boxes, ((0, 0), (0, pad), (0, 0)))
    bx = jnp.transpose(bx, (0, 2, 1))
    scp = jnp.pad(scores, ((0, 0), (0, pad)))[:, None, :]
    lbp = jnp.pad(labels, ((0, 0), (0, pad))).astype(f32)[:, None, :]
    idx = jnp.broadcast_to(jnp.arange(_NP, dtype=f32), (b, 1, _NP))
    one = jnp.ones((b, 1, _NP), f32)
    feat = jnp.concatenate([bx, scp, lbp, idx, one],
                           axis=1).reshape(b, 9, _NB, 128)

    grid = (b,)
    out = pl.pallas_call(
        _body,
        grid=grid,
        in_specs=[
            pl.BlockSpec((1, 9, _NB, 128), lambda i: (i, 0, 0, 0)),
            pl.BlockSpec((1, _NB, 128), lambda i: (i, 0, 0)),
        ],
        out_specs=[
            pl.BlockSpec((1, _DET, 5), lambda i: (i, 0, 0)),
            pl.BlockSpec((1, 1, _DET), lambda i: (i, 0, 0)),
            pl.BlockSpec((1, 1, _DET), lambda i: (i, 0, 0)),
        ],
        out_shape=[
            jax.ShapeDtypeStruct((b, _DET, 5), f32),
            jax.ShapeDtypeStruct((b, 1, _DET), f32),
            jax.ShapeDtypeStruct((b, 1, _DET), jnp.int32),
        ],
        scratch_shapes=[
            pltpu.VMEM((_KP, _KP), f32),
            pltpu.VMEM((_KP, _KP), f32),
            pltpu.VMEM((_TPAD, 9), f32),
            pltpu.VMEM((_NB, 128), f32),
            pltpu.VMEM((_NB, 1), f32),
            pltpu.VMEM((1, _KP), f32),
        ],
    )(feat, sc)
    return out[0], out[1].reshape(b, _DET), out[2].reshape(b, _DET)
